# baseline (speedup 1.0000x reference)
import functools

import jax
import jax.numpy as jnp
from jax import lax
from jax.experimental import pallas as pl
from jax.experimental.pallas import tpu as pltpu
from jax.experimental.pallas import tpu_sc as plsc

INIT_SIZE = 8192
EMB_DIM = 1024
ORIGIN_SHIFT = INIT_SIZE // 2 + 1

NUM_SC_CORES = 2
NUM_SUBCORES = 16
NUM_WORKERS = NUM_SC_CORES * NUM_SUBCORES

CHUNK = 32


def _sc_row_range_copy(embed, n_rows, start_row):
    emb_dim = embed.shape[1]
    rows_per_w = n_rows // NUM_WORKERS
    n_chunks = rows_per_w // CHUNK
    assert rows_per_w * NUM_WORKERS == n_rows
    assert n_chunks * CHUNK == rows_per_w

    mesh = plsc.VectorSubcoreMesh(core_axis_name="c", subcore_axis_name="s")
    chunk_elems = CHUNK * emb_dim

    @functools.partial(
        pl.kernel,
        mesh=mesh,
        out_type=jax.ShapeDtypeStruct((n_rows * emb_dim,), embed.dtype),
        scratch_types=[
            pltpu.VMEM((chunk_elems,), embed.dtype),
            pltpu.VMEM((chunk_elems,), embed.dtype),
            pltpu.SemaphoreType.DMA,
            pltpu.SemaphoreType.DMA,
            pltpu.SemaphoreType.DMA,
            pltpu.SemaphoreType.DMA,
        ],
    )
    def body(embed_hbm, out_hbm, buf0, buf1, si0, si1, so0, so1):
        wid = lax.axis_index("s") * NUM_SC_CORES + lax.axis_index("c")
        base = wid * rows_per_w * emb_dim
        src0 = base + start_row * emb_dim
        bufs = (buf0, buf1)
        sin = (si0, si1)
        sout = (so0, so1)

        def in_copy(i):
            return pltpu.make_async_copy(
                embed_hbm.at[pl.ds(src0 + i * chunk_elems, chunk_elems)],
                bufs[i % 2], sin[i % 2])

        def out_copy(i):
            return pltpu.make_async_copy(
                bufs[i % 2], out_hbm.at[pl.ds(base + i * chunk_elems, chunk_elems)],
                sout[i % 2])

        for i in range(min(2, n_chunks)):
            in_copy(i).start()
        outs = []
        for i in range(n_chunks):
            in_copy(i).wait()
            oc = out_copy(i)
            oc.start()
            if i + 2 < n_chunks:
                oc.wait()
                in_copy(i + 2).start()
            else:
                outs.append(oc)
        for oc in outs:
            oc.wait()

    out_flat = body(embed.reshape(-1))
    return out_flat.reshape(n_rows, emb_dim)


def kernel(input, embed):
    bsz, seq_len = input.shape
    n_rows = 2 * seq_len
    start_row = ORIGIN_SHIFT - seq_len
    return _sc_row_range_copy(embed, n_rows, start_row)
